# baseline (speedup 1.0000x reference)
import functools

import jax
import jax.numpy as jnp
from jax import lax
from jax.experimental import pallas as pl
from jax.experimental.pallas import tpu as pltpu
from jax.experimental.pallas import tpu_sc as plsc

_N = 10000
_E = 160000
_DIN = 128
_DE = 16
_NF = 32
_NH = 8
_CW = 16

_CH = 1000
_NCH = _E // _CH
_NW = 32
_GT = _NCH // _NW
_RPT = _N // 16
_CPS = (_E // 2) // _CH
_ST = _CPS // 16

_mesh = plsc.VectorSubcoreMesh(core_axis_name="c", subcore_axis_name="s")
_SC_PARAMS = pltpu.CompilerParams(use_tc_tiling_on_sc=False)

def _dot(a, b):
    return jnp.dot(a, b, preferred_element_type=jnp.float32)


def _node_proj_body(h_ref, w0_ref, b0_ref, o_ref):
    o_ref[...] = jnp.maximum(_dot(h_ref[...], w0_ref[...]) + b0_ref[...], 0.0)


def _node_proj(h, w0, b0r):
    bn = 2000
    return pl.pallas_call(
        _node_proj_body,
        grid=(_N // bn,),
        in_specs=[
            pl.BlockSpec((bn, _DIN), lambda i: (i, 0)),
            pl.BlockSpec((_DIN, _NF), lambda i: (0, 0)),
            pl.BlockSpec((1, _NF), lambda i: (0, 0)),
        ],
        out_specs=pl.BlockSpec((bn, _NF), lambda i: (i, 0)),
        out_shape=jax.ShapeDtypeStruct((_N, _NF), jnp.float32),
    )(h, w0, b0r)


def _edge_mlp_body(ea4_ref, ws_ref, bs_ref, w1_ref, b1_ref, hid_ref):
    t = jnp.maximum(_dot(ea4_ref[...], ws_ref[...]) + bs_ref[...], 0.0)
    hid_ref[...] = jnp.maximum(_dot(t, w1_ref[...]) + b1_ref[...], 0.0)


def _edge_mlp(ea4, wsbd, bs4, w1bd, b14):
    be = 4000
    e4 = _E // 4
    return pl.pallas_call(
        _edge_mlp_body,
        grid=(e4 // be,),
        in_specs=[
            pl.BlockSpec((be, 4 * _DE), lambda i: (i, 0)),
            pl.BlockSpec((4 * _DE, 4 * _NF), lambda i: (0, 0)),
            pl.BlockSpec((1, 4 * _NF), lambda i: (0, 0)),
            pl.BlockSpec((4 * _NF, 4 * _NH), lambda i: (0, 0)),
            pl.BlockSpec((1, 4 * _NH), lambda i: (0, 0)),
        ],
        out_specs=pl.BlockSpec((be, 4 * _NH), lambda i: (i, 0)),
        out_shape=jax.ShapeDtypeStruct((e4, 4 * _NH), jnp.float32),
    )(ea4, wsbd, bs4, w1bd, b14)


def _msg_body(xjp_ref, hid4_ref, w2a_ref, w2b_ref, r4_ref, s4_ref, msgp_ref):
    bf = jnp.bfloat16
    xjp = xjp_ref[...].astype(bf)
    ya = _dot(xjp, w2a_ref[...].astype(bf))
    yb = _dot(xjp, w2b_ref[...].astype(bf))
    hidx = _dot(hid4_ref[...].astype(bf),
                r4_ref[...].astype(bf))
    prod = (hidx * ya).astype(bf)
    msgp_ref[...] = _dot(prod, s4_ref[...].astype(bf)) + yb


def _msg(xjp, hid4, w2a, w2b, r4, s4):
    bt4 = 1000
    e4 = _E // 4
    return pl.pallas_call(
        _msg_body,
        grid=(e4 // bt4,),
        in_specs=[
            pl.BlockSpec((bt4, 128), lambda i: (i, 0)),
            pl.BlockSpec((bt4, 4 * _NH), lambda i: (i, 0)),
            pl.BlockSpec((128, 4 * _NH * _NF), lambda i: (0, 0)),
            pl.BlockSpec((128, 128), lambda i: (0, 0)),
            pl.BlockSpec((4 * _NH, 4 * _NH * _NF), lambda i: (0, 0)),
            pl.BlockSpec((4 * _NH * _NF, 128), lambda i: (0, 0)),
        ],
        out_specs=pl.BlockSpec((bt4, 128), lambda i: (i, 0)),
        out_shape=jax.ShapeDtypeStruct((e4, 128), jnp.float32),
    )(xjp, hid4, w2a, w2b, r4, s4)


def _update_body(agg2_ref, cnt2_ref, out_ref, root_ref, bias_ref,
                 wih_ref, bih_ref, whh_ref, bhh_ref, new_ref):
    cnt = jnp.maximum(cnt2_ref[0, :, 0:1] + cnt2_ref[1, :, 0:1], 1.0)
    agg = (agg2_ref[0] + agg2_ref[1]) / cnt
    out = out_ref[...]
    m = jnp.maximum(agg + _dot(out, root_ref[...]) + bias_ref[...], 0.0)
    gi = _dot(m, wih_ref[...]) + bih_ref[...]
    gh = _dot(out, whh_ref[...]) + bhh_ref[...]
    r = jax.nn.sigmoid(gi[:, :_NF] + gh[:, :_NF])
    z = jax.nn.sigmoid(gi[:, _NF:2 * _NF] + gh[:, _NF:2 * _NF])
    n = jnp.tanh(gi[:, 2 * _NF:] + r * gh[:, 2 * _NF:])
    new_ref[...] = (1.0 - z) * n + z * out


def _update(agg2, cnt2, out, root, biasr, wih_t, bihr, whh_t, bhhr):
    bn = 2000
    return pl.pallas_call(
        _update_body,
        grid=(_N // bn,),
        in_specs=[
            pl.BlockSpec((2, bn, _NF), lambda i: (0, i, 0)),
            pl.BlockSpec((2, bn, _CW), lambda i: (0, i, 0)),
            pl.BlockSpec((bn, _NF), lambda i: (i, 0)),
            pl.BlockSpec((_NF, _NF), lambda i: (0, 0)),
            pl.BlockSpec((1, _NF), lambda i: (0, 0)),
            pl.BlockSpec((_NF, 3 * _NF), lambda i: (0, 0)),
            pl.BlockSpec((1, 3 * _NF), lambda i: (0, 0)),
            pl.BlockSpec((_NF, 3 * _NF), lambda i: (0, 0)),
            pl.BlockSpec((1, 3 * _NF), lambda i: (0, 0)),
        ],
        out_specs=pl.BlockSpec((bn, _NF), lambda i: (i, 0)),
        out_shape=jax.ShapeDtypeStruct((_N, _NF), jnp.float32),
    )(agg2, cnt2, out, root, biasr, wih_t, bihr, whh_t, bhhr)


def _sc_gather(table, src):
    @functools.partial(
        pl.kernel,
        mesh=_mesh,
        compiler_params=_SC_PARAMS,
        out_type=jax.ShapeDtypeStruct((_E, _NF), jnp.float32),
        scratch_types=[
            pltpu.VMEM((_CH,), jnp.int32),
            pltpu.VMEM((_CH,), jnp.int32),
            pltpu.VMEM((_CH, _NF), jnp.float32),
            pltpu.VMEM((_CH, _NF), jnp.float32),
            pltpu.SemaphoreType.DMA,
            pltpu.SemaphoreType.DMA,
            pltpu.SemaphoreType.DMA,
            pltpu.SemaphoreType.DMA,
            pltpu.SemaphoreType.DMA,
            pltpu.SemaphoreType.DMA,
        ],
    )
    def k(table_hbm, src_hbm, xj_hbm, i0, i1, r0, r1,
          si0, si1, sg0, sg1, ss0, ss1):
        wid = lax.axis_index("s") * 2 + lax.axis_index("c")
        idx = [i0, i1]
        rows = [r0, r1]
        sis = [si0, si1]
        sgs = [sg0, sg1]
        sss = [ss0, ss1]

        def base(j):
            return (wid + j * _NW) * _CH

        hi = [None] * _GT
        hs = [None] * _GT
        hi[0] = pltpu.async_copy(src_hbm.at[pl.ds(base(0), _CH)], idx[0],
                                 sis[0])
        for j in range(_GT):
            if j + 1 < _GT:
                b = (j + 1) % 2
                hi[j + 1] = pltpu.async_copy(
                    src_hbm.at[pl.ds(base(j + 1), _CH)], idx[b], sis[b])
            hi[j].wait()
            if j >= 2:
                hs[j - 2].wait()
            pltpu.async_copy(table_hbm.at[idx[j % 2]], rows[j % 2],
                             sgs[j % 2]).wait()
            hs[j] = pltpu.async_copy(rows[j % 2],
                                     xj_hbm.at[pl.ds(base(j), _CH)],
                                     sss[j % 2])
        hs[_GT - 2].wait()
        hs[_GT - 1].wait()

    return k(table, src)


def _make_sc_scatter(with_counts):
    outs = [jax.ShapeDtypeStruct((2, _N, _NF), jnp.float32)]
    scratch = [
        pltpu.VMEM((_CH,), jnp.int32),
        pltpu.VMEM((_CH,), jnp.int32),
        pltpu.VMEM((_CH, _NF), jnp.float32),
        pltpu.VMEM((_CH, _NF), jnp.float32),
        pltpu.VMEM_SHARED((_N, _NF), jnp.float32),
        pltpu.SemaphoreType.DMA,
        pltpu.SemaphoreType.DMA,
        pltpu.SemaphoreType.DMA,
        pltpu.SemaphoreType.DMA,
    ]
    if with_counts:
        outs.append(jax.ShapeDtypeStruct((2, _N, _CW), jnp.float32))
        scratch.append(pltpu.VMEM((_CH, _CW), jnp.float32))
        scratch.append(pltpu.VMEM_SHARED((_N, _CW), jnp.float32))

    def body(*refs):
        if with_counts:
            (msg_hbm, dst_hbm, z32_hbm, z16_hbm, ones_hbm, agg2_hbm,
             cnt2_hbm, i0, i1, v0, v1, agg_s, si0, si1, sv0, sv1,
             ones_v, cnt_s) = refs
        else:
            (msg_hbm, dst_hbm, z32_hbm, z16_hbm, ones_hbm, agg2_hbm,
             i0, i1, v0, v1, agg_s, si0, si1, sv0, sv1) = refs
        cid = lax.axis_index("c")
        sid = lax.axis_index("s")
        row0 = sid * _RPT
        idx = [i0, i1]
        val = [v0, v1]
        sis = [si0, si1]
        svs = [sv0, sv1]

        def base(j):
            return (cid * _CPS + sid + j * 16) * _CH

        pltpu.sync_copy(z32_hbm, agg_s.at[pl.ds(row0, _RPT)])
        if with_counts:
            pltpu.sync_copy(z16_hbm, cnt_s.at[pl.ds(row0, _RPT)])
            pltpu.sync_copy(ones_hbm, ones_v)
        plsc.subcore_barrier()

        hi = [None] * _ST
        hv = [None] * _ST
        hi[0] = pltpu.async_copy(dst_hbm.at[pl.ds(base(0), _CH)], idx[0],
                                 sis[0])
        hv[0] = pltpu.async_copy(msg_hbm.at[pl.ds(base(0), _CH)], val[0],
                                 svs[0])
        for j in range(_ST):
            if j + 1 < _ST:
                b = (j + 1) % 2
                hi[j + 1] = pltpu.async_copy(
                    dst_hbm.at[pl.ds(base(j + 1), _CH)], idx[b], sis[b])
                hv[j + 1] = pltpu.async_copy(
                    msg_hbm.at[pl.ds(base(j + 1), _CH)], val[b], svs[b])
            hi[j].wait()
            hv[j].wait()
            pltpu.sync_copy(val[j % 2], agg_s.at[idx[j % 2]], add=True)
            if with_counts:
                pltpu.sync_copy(ones_v, cnt_s.at[idx[j % 2]], add=True)

        plsc.subcore_barrier()
        pltpu.sync_copy(agg_s.at[pl.ds(row0, _RPT)],
                        agg2_hbm.at[cid, pl.ds(row0, _RPT)])
        if with_counts:
            pltpu.sync_copy(cnt_s.at[pl.ds(row0, _RPT)],
                            cnt2_hbm.at[cid, pl.ds(row0, _RPT)])

    return functools.partial(
        pl.kernel, mesh=_mesh,
        compiler_params=_SC_PARAMS,
        out_type=tuple(outs) if with_counts else outs[0],
        scratch_types=scratch,
    )(body)


_sc_scatter_counts = _make_sc_scatter(True)
_sc_scatter = _make_sc_scatter(False)


def kernel(h, edge_index, edge_weight, edge_attr, W0, b0, Ws, bs,
           W1, b1, W2, b2, root, bias, W_ih, W_hh, b_ih, b_hh):
    src = edge_index[0]
    dst = edge_index[1]

    eye4 = jnp.eye(4, dtype=jnp.float32)
    w2core = W2.reshape(_NH, _NF, _NF).transpose(1, 0, 2).reshape(
        _NF, _NH * _NF)
    w2a = jnp.kron(eye4, w2core)
    w2b = jnp.kron(eye4, b2.reshape(_NF, _NF))
    rmat = jnp.kron(jnp.eye(_NH, dtype=jnp.float32),
                    jnp.ones((1, _NF), jnp.float32))
    smat = jnp.kron(jnp.ones((_NH, 1), jnp.float32),
                    jnp.eye(_NF, dtype=jnp.float32))
    r4 = jnp.kron(eye4, rmat)
    s4 = jnp.kron(eye4, smat)
    wsbd = jnp.kron(eye4, Ws)
    w1bd = jnp.kron(eye4, W1)
    bs4 = jnp.tile(bs, 4).reshape(1, -1)
    b14 = jnp.tile(b1, 4).reshape(1, -1)
    ea4 = edge_attr.reshape(_E // 4, 4 * _DE)
    wih_t = W_ih.T
    whh_t = W_hh.T
    b0r = b0.reshape(1, -1)
    bsr = bs.reshape(1, -1)
    b1r = b1.reshape(1, -1)
    biasr = bias.reshape(1, -1)
    bihr = b_ih.reshape(1, -1)
    bhhr = b_hh.reshape(1, -1)
    z32 = jnp.zeros((_RPT, _NF), jnp.float32)
    z16 = jnp.zeros((_RPT, _CW), jnp.float32)
    o16 = jnp.ones((_CH, _CW), jnp.float32)

    out = _node_proj(h, W0, b0r)
    hid4 = _edge_mlp(ea4, wsbd, bs4, w1bd, b14)

    cnt2 = None
    for it in range(2):
        xjp = _sc_gather(out, src).reshape(_E // 4, 128)
        msg = _msg(xjp, hid4, w2a, w2b, r4, s4).reshape(_E, _NF)
        if it == 0:
            agg2, cnt2 = _sc_scatter_counts(msg, dst, z32, z16, o16)
        else:
            agg2 = _sc_scatter(msg, dst, z32, z16, o16)
        out = _update(agg2, cnt2, out, root, biasr, wih_t, bihr, whh_t, bhhr)
    return out


# speedup vs baseline: 1.1495x; 1.1495x over previous
import functools

import jax
import jax.numpy as jnp
from jax import lax
from jax.experimental import pallas as pl
from jax.experimental.pallas import tpu as pltpu
from jax.experimental.pallas import tpu_sc as plsc

_N = 10000
_E = 160000
_DIN = 128
_DE = 16
_NF = 32
_NH = 8
_CW = 16

_CH = 1000
_NCH = _E // _CH
_NW = 32
_GT = _NCH // _NW
_RPT = _N // 16
_CPS = (_E // 2) // _CH
_ST = _CPS // 16

_mesh = plsc.VectorSubcoreMesh(core_axis_name="c", subcore_axis_name="s")
_SC_PARAMS = pltpu.CompilerParams(use_tc_tiling_on_sc=False)

def _dot(a, b):
    return jnp.dot(a, b, preferred_element_type=jnp.float32)


def _node_proj_body(h_ref, w0_ref, b0_ref, o_ref):
    o_ref[...] = jnp.maximum(_dot(h_ref[...], w0_ref[...]) + b0_ref[...], 0.0)


def _node_proj(h, w0, b0r):
    bn = 2000
    return pl.pallas_call(
        _node_proj_body,
        grid=(_N // bn,),
        in_specs=[
            pl.BlockSpec((bn, _DIN), lambda i: (i, 0)),
            pl.BlockSpec((_DIN, _NF), lambda i: (0, 0)),
            pl.BlockSpec((1, _NF), lambda i: (0, 0)),
        ],
        out_specs=pl.BlockSpec((bn, _NF), lambda i: (i, 0)),
        out_shape=jax.ShapeDtypeStruct((_N, _NF), jnp.float32),
    )(h, w0, b0r)


def _edge_mlp_body(ea4_ref, ws_ref, bs_ref, w1_ref, b1_ref, hid_ref):
    t = jnp.maximum(_dot(ea4_ref[...], ws_ref[...]) + bs_ref[...], 0.0)
    hid_ref[...] = jnp.maximum(_dot(t, w1_ref[...]) + b1_ref[...], 0.0)


def _edge_mlp(ea4, wsbd, bs4, w1bd, b14):
    be = 4000
    e4 = _E // 4
    return pl.pallas_call(
        _edge_mlp_body,
        grid=(e4 // be,),
        in_specs=[
            pl.BlockSpec((be, 4 * _DE), lambda i: (i, 0)),
            pl.BlockSpec((4 * _DE, 4 * _NF), lambda i: (0, 0)),
            pl.BlockSpec((1, 4 * _NF), lambda i: (0, 0)),
            pl.BlockSpec((4 * _NF, 4 * _NH), lambda i: (0, 0)),
            pl.BlockSpec((1, 4 * _NH), lambda i: (0, 0)),
        ],
        out_specs=pl.BlockSpec((be, 4 * _NH), lambda i: (i, 0)),
        out_shape=jax.ShapeDtypeStruct((e4, 4 * _NH), jnp.float32),
    )(ea4, wsbd, bs4, w1bd, b14)


def _msg_body(xjp_ref, hid4_ref, w2a_ref, w2b_ref, r4_ref, s4_ref, msgp_ref):
    xjp = xjp_ref[...]
    ya = _dot(xjp, w2a_ref[...])
    yb = _dot(xjp, w2b_ref[...])
    hidx = _dot(hid4_ref[...], r4_ref[...])
    msgp_ref[...] = _dot(hidx * ya, s4_ref[...]) + yb


def _msg(xjp, hid4, w2a, w2b, r4, s4):
    bt4 = 2000
    e4 = _E // 4
    return pl.pallas_call(
        _msg_body,
        grid=(e4 // bt4,),
        in_specs=[
            pl.BlockSpec((bt4, 128), lambda i: (i, 0)),
            pl.BlockSpec((bt4, 4 * _NH), lambda i: (i, 0)),
            pl.BlockSpec((128, 4 * _NH * _NF), lambda i: (0, 0)),
            pl.BlockSpec((128, 128), lambda i: (0, 0)),
            pl.BlockSpec((4 * _NH, 4 * _NH * _NF), lambda i: (0, 0)),
            pl.BlockSpec((4 * _NH * _NF, 128), lambda i: (0, 0)),
        ],
        out_specs=pl.BlockSpec((bt4, 128), lambda i: (i, 0)),
        out_shape=jax.ShapeDtypeStruct((e4, 128), jnp.float32),
    )(xjp, hid4, w2a, w2b, r4, s4)


def _update_body(agg2_ref, cnt2_ref, out_ref, root_ref, bias_ref,
                 wih_ref, bih_ref, whh_ref, bhh_ref, new_ref):
    cnt = jnp.maximum(cnt2_ref[0, :, 0:1] + cnt2_ref[1, :, 0:1], 1.0)
    agg = (agg2_ref[0] + agg2_ref[1]) / cnt
    out = out_ref[...]
    m = jnp.maximum(agg + _dot(out, root_ref[...]) + bias_ref[...], 0.0)
    gi = _dot(m, wih_ref[...]) + bih_ref[...]
    gh = _dot(out, whh_ref[...]) + bhh_ref[...]
    r = jax.nn.sigmoid(gi[:, :_NF] + gh[:, :_NF])
    z = jax.nn.sigmoid(gi[:, _NF:2 * _NF] + gh[:, _NF:2 * _NF])
    n = jnp.tanh(gi[:, 2 * _NF:] + r * gh[:, 2 * _NF:])
    new_ref[...] = (1.0 - z) * n + z * out


def _update(agg2, cnt2, out, root, biasr, wih_t, bihr, whh_t, bhhr):
    bn = 2000
    return pl.pallas_call(
        _update_body,
        grid=(_N // bn,),
        in_specs=[
            pl.BlockSpec((2, bn, _NF), lambda i: (0, i, 0)),
            pl.BlockSpec((2, bn, _CW), lambda i: (0, i, 0)),
            pl.BlockSpec((bn, _NF), lambda i: (i, 0)),
            pl.BlockSpec((_NF, _NF), lambda i: (0, 0)),
            pl.BlockSpec((1, _NF), lambda i: (0, 0)),
            pl.BlockSpec((_NF, 3 * _NF), lambda i: (0, 0)),
            pl.BlockSpec((1, 3 * _NF), lambda i: (0, 0)),
            pl.BlockSpec((_NF, 3 * _NF), lambda i: (0, 0)),
            pl.BlockSpec((1, 3 * _NF), lambda i: (0, 0)),
        ],
        out_specs=pl.BlockSpec((bn, _NF), lambda i: (i, 0)),
        out_shape=jax.ShapeDtypeStruct((_N, _NF), jnp.float32),
    )(agg2, cnt2, out, root, biasr, wih_t, bihr, whh_t, bhhr)


def _sc_gather(table, src):
    @functools.partial(
        pl.kernel,
        mesh=_mesh,
        compiler_params=_SC_PARAMS,
        out_type=jax.ShapeDtypeStruct((_E, _NF), jnp.float32),
        scratch_types=[
            pltpu.VMEM((_CH,), jnp.int32),
            pltpu.VMEM((_CH,), jnp.int32),
            pltpu.VMEM((_CH, _NF), jnp.float32),
            pltpu.VMEM((_CH, _NF), jnp.float32),
            pltpu.SemaphoreType.DMA,
            pltpu.SemaphoreType.DMA,
            pltpu.SemaphoreType.DMA,
            pltpu.SemaphoreType.DMA,
            pltpu.SemaphoreType.DMA,
            pltpu.SemaphoreType.DMA,
        ],
    )
    def k(table_hbm, src_hbm, xj_hbm, i0, i1, r0, r1,
          si0, si1, sg0, sg1, ss0, ss1):
        wid = lax.axis_index("s") * 2 + lax.axis_index("c")
        idx = [i0, i1]
        rows = [r0, r1]
        sis = [si0, si1]
        sgs = [sg0, sg1]
        sss = [ss0, ss1]

        def base(j):
            return (wid + j * _NW) * _CH

        hi = [None] * _GT
        hs = [None] * _GT
        hi[0] = pltpu.async_copy(src_hbm.at[pl.ds(base(0), _CH)], idx[0],
                                 sis[0])
        for j in range(_GT):
            if j + 1 < _GT:
                b = (j + 1) % 2
                hi[j + 1] = pltpu.async_copy(
                    src_hbm.at[pl.ds(base(j + 1), _CH)], idx[b], sis[b])
            hi[j].wait()
            if j >= 2:
                hs[j - 2].wait()
            pltpu.async_copy(table_hbm.at[idx[j % 2]], rows[j % 2],
                             sgs[j % 2]).wait()
            hs[j] = pltpu.async_copy(rows[j % 2],
                                     xj_hbm.at[pl.ds(base(j), _CH)],
                                     sss[j % 2])
        hs[_GT - 2].wait()
        hs[_GT - 1].wait()

    return k(table, src)


def _make_sc_scatter(with_counts):
    outs = [jax.ShapeDtypeStruct((2, _N, _NF), jnp.float32)]
    scratch = [
        pltpu.VMEM((_CH,), jnp.int32),
        pltpu.VMEM((_CH,), jnp.int32),
        pltpu.VMEM((_CH, _NF), jnp.float32),
        pltpu.VMEM((_CH, _NF), jnp.float32),
        pltpu.VMEM_SHARED((_N, _NF), jnp.float32),
        pltpu.SemaphoreType.DMA,
        pltpu.SemaphoreType.DMA,
        pltpu.SemaphoreType.DMA,
        pltpu.SemaphoreType.DMA,
    ]
    if with_counts:
        outs.append(jax.ShapeDtypeStruct((2, _N, _CW), jnp.float32))
        scratch.append(pltpu.VMEM((_CH, _CW), jnp.float32))
        scratch.append(pltpu.VMEM_SHARED((_N, _CW), jnp.float32))

    def body(*refs):
        if with_counts:
            (msg_hbm, dst_hbm, z32_hbm, z16_hbm, ones_hbm, agg2_hbm,
             cnt2_hbm, i0, i1, v0, v1, agg_s, si0, si1, sv0, sv1,
             ones_v, cnt_s) = refs
        else:
            (msg_hbm, dst_hbm, z32_hbm, z16_hbm, ones_hbm, agg2_hbm,
             i0, i1, v0, v1, agg_s, si0, si1, sv0, sv1) = refs
        cid = lax.axis_index("c")
        sid = lax.axis_index("s")
        row0 = sid * _RPT
        idx = [i0, i1]
        val = [v0, v1]
        sis = [si0, si1]
        svs = [sv0, sv1]

        def base(j):
            return (cid * _CPS + sid + j * 16) * _CH

        pltpu.sync_copy(z32_hbm, agg_s.at[pl.ds(row0, _RPT)])
        if with_counts:
            pltpu.sync_copy(z16_hbm, cnt_s.at[pl.ds(row0, _RPT)])
            pltpu.sync_copy(ones_hbm, ones_v)
        plsc.subcore_barrier()

        hi = [None] * _ST
        hv = [None] * _ST
        hi[0] = pltpu.async_copy(dst_hbm.at[pl.ds(base(0), _CH)], idx[0],
                                 sis[0])
        hv[0] = pltpu.async_copy(msg_hbm.at[pl.ds(base(0), _CH)], val[0],
                                 svs[0])
        for j in range(_ST):
            if j + 1 < _ST:
                b = (j + 1) % 2
                hi[j + 1] = pltpu.async_copy(
                    dst_hbm.at[pl.ds(base(j + 1), _CH)], idx[b], sis[b])
                hv[j + 1] = pltpu.async_copy(
                    msg_hbm.at[pl.ds(base(j + 1), _CH)], val[b], svs[b])
            hi[j].wait()
            hv[j].wait()
            pltpu.sync_copy(val[j % 2], agg_s.at[idx[j % 2]], add=True)
            if with_counts:
                pltpu.sync_copy(ones_v, cnt_s.at[idx[j % 2]], add=True)

        plsc.subcore_barrier()
        pltpu.sync_copy(agg_s.at[pl.ds(row0, _RPT)],
                        agg2_hbm.at[cid, pl.ds(row0, _RPT)])
        if with_counts:
            pltpu.sync_copy(cnt_s.at[pl.ds(row0, _RPT)],
                            cnt2_hbm.at[cid, pl.ds(row0, _RPT)])

    return functools.partial(
        pl.kernel, mesh=_mesh,
        compiler_params=_SC_PARAMS,
        out_type=tuple(outs) if with_counts else outs[0],
        scratch_types=scratch,
    )(body)


_sc_scatter_counts = _make_sc_scatter(True)
_sc_scatter = _make_sc_scatter(False)


def kernel(h, edge_index, edge_weight, edge_attr, W0, b0, Ws, bs,
           W1, b1, W2, b2, root, bias, W_ih, W_hh, b_ih, b_hh):
    src = edge_index[0]
    dst = edge_index[1]

    eye4 = jnp.eye(4, dtype=jnp.float32)
    w2core = W2.reshape(_NH, _NF, _NF).transpose(1, 0, 2).reshape(
        _NF, _NH * _NF)
    w2a = jnp.kron(eye4, w2core)
    w2b = jnp.kron(eye4, b2.reshape(_NF, _NF))
    rmat = jnp.kron(jnp.eye(_NH, dtype=jnp.float32),
                    jnp.ones((1, _NF), jnp.float32))
    smat = jnp.kron(jnp.ones((_NH, 1), jnp.float32),
                    jnp.eye(_NF, dtype=jnp.float32))
    r4 = jnp.kron(eye4, rmat)
    s4 = jnp.kron(eye4, smat)
    wsbd = jnp.kron(eye4, Ws)
    w1bd = jnp.kron(eye4, W1)
    bs4 = jnp.tile(bs, 4).reshape(1, -1)
    b14 = jnp.tile(b1, 4).reshape(1, -1)
    ea4 = edge_attr.reshape(_E // 4, 4 * _DE)
    wih_t = W_ih.T
    whh_t = W_hh.T
    b0r = b0.reshape(1, -1)
    bsr = bs.reshape(1, -1)
    b1r = b1.reshape(1, -1)
    biasr = bias.reshape(1, -1)
    bihr = b_ih.reshape(1, -1)
    bhhr = b_hh.reshape(1, -1)
    z32 = jnp.zeros((_RPT, _NF), jnp.float32)
    z16 = jnp.zeros((_RPT, _CW), jnp.float32)
    o16 = jnp.ones((_CH, _CW), jnp.float32)

    out = _node_proj(h, W0, b0r)
    hid4 = _edge_mlp(ea4, wsbd, bs4, w1bd, b14)

    cnt2 = None
    for it in range(2):
        xjp = _sc_gather(out, src).reshape(_E // 4, 128)
        msg = _msg(xjp, hid4, w2a, w2b, r4, s4).reshape(_E, _NF)
        if it == 0:
            agg2, cnt2 = _sc_scatter_counts(msg, dst, z32, z16, o16)
        else:
            agg2 = _sc_scatter(msg, dst, z32, z16, o16)
        out = _update(agg2, cnt2, out, root, biasr, wih_t, bihr, whh_t, bhhr)
    return out


# speedup vs baseline: 1.1637x; 1.0124x over previous
import functools

import jax
import jax.numpy as jnp
from jax import lax
from jax.experimental import pallas as pl
from jax.experimental.pallas import tpu as pltpu
from jax.experimental.pallas import tpu_sc as plsc

_N = 10000
_E = 160000
_DIN = 128
_DE = 16
_NF = 32
_NH = 8
_CW = 16

_CH = 1000
_NCH = _E // _CH
_NW = 32
_GT = _NCH // _NW
_RPT = _N // 16
_CPS = (_E // 2) // _CH
_ST = _CPS // 16

_mesh = plsc.VectorSubcoreMesh(core_axis_name="c", subcore_axis_name="s")
_SC_PARAMS = pltpu.CompilerParams(use_tc_tiling_on_sc=False)

def _dot(a, b):
    return jnp.dot(a, b, preferred_element_type=jnp.float32)


def _node_proj_body(h_ref, w0_ref, b0_ref, o_ref):
    o_ref[...] = jnp.maximum(_dot(h_ref[...], w0_ref[...]) + b0_ref[...], 0.0)


def _node_proj(h, w0, b0r):
    bn = 2000
    return pl.pallas_call(
        _node_proj_body,
        grid=(_N // bn,),
        in_specs=[
            pl.BlockSpec((bn, _DIN), lambda i: (i, 0)),
            pl.BlockSpec((_DIN, _NF), lambda i: (0, 0)),
            pl.BlockSpec((1, _NF), lambda i: (0, 0)),
        ],
        out_specs=pl.BlockSpec((bn, _NF), lambda i: (i, 0)),
        out_shape=jax.ShapeDtypeStruct((_N, _NF), jnp.float32),
    )(h, w0, b0r)


def _msg_body(xjp_ref, ea4_ref, ws_ref, bs_ref, w1_ref, b1_ref,
              w2a_ref, w2b_ref, r4_ref, s4_ref, msgp_ref):
    t4 = jnp.maximum(_dot(ea4_ref[...], ws_ref[...]) + bs_ref[...], 0.0)
    hid4 = jnp.maximum(_dot(t4, w1_ref[...]) + b1_ref[...], 0.0)
    xjp = xjp_ref[...]
    ya = _dot(xjp, w2a_ref[...])
    yb = _dot(xjp, w2b_ref[...])
    hidx = _dot(hid4, r4_ref[...])
    msgp_ref[...] = _dot(hidx * ya, s4_ref[...]) + yb


def _msg(xjp, ea4, wsbd, bs4, w1bd, b14, w2a, w2b, r4, s4):
    bt4 = 2000
    e4 = _E // 4
    return pl.pallas_call(
        _msg_body,
        grid=(e4 // bt4,),
        in_specs=[
            pl.BlockSpec((bt4, 128), lambda i: (i, 0)),
            pl.BlockSpec((bt4, 4 * _DE), lambda i: (i, 0)),
            pl.BlockSpec((4 * _DE, 4 * _NF), lambda i: (0, 0)),
            pl.BlockSpec((1, 4 * _NF), lambda i: (0, 0)),
            pl.BlockSpec((4 * _NF, 4 * _NH), lambda i: (0, 0)),
            pl.BlockSpec((1, 4 * _NH), lambda i: (0, 0)),
            pl.BlockSpec((128, 4 * _NH * _NF), lambda i: (0, 0)),
            pl.BlockSpec((128, 128), lambda i: (0, 0)),
            pl.BlockSpec((4 * _NH, 4 * _NH * _NF), lambda i: (0, 0)),
            pl.BlockSpec((4 * _NH * _NF, 128), lambda i: (0, 0)),
        ],
        out_specs=pl.BlockSpec((bt4, 128), lambda i: (i, 0)),
        out_shape=jax.ShapeDtypeStruct((e4, 128), jnp.float32),
    )(xjp, ea4, wsbd, bs4, w1bd, b14, w2a, w2b, r4, s4)


def _update_body(agg2_ref, cnt2_ref, out_ref, root_ref, bias_ref,
                 wih_ref, bih_ref, whh_ref, bhh_ref, new_ref):
    cnt = jnp.maximum(cnt2_ref[0, :, 0:1] + cnt2_ref[1, :, 0:1], 1.0)
    agg = (agg2_ref[0] + agg2_ref[1]) / cnt
    out = out_ref[...]
    m = jnp.maximum(agg + _dot(out, root_ref[...]) + bias_ref[...], 0.0)
    gi = _dot(m, wih_ref[...]) + bih_ref[...]
    gh = _dot(out, whh_ref[...]) + bhh_ref[...]
    r = jax.nn.sigmoid(gi[:, :_NF] + gh[:, :_NF])
    z = jax.nn.sigmoid(gi[:, _NF:2 * _NF] + gh[:, _NF:2 * _NF])
    n = jnp.tanh(gi[:, 2 * _NF:] + r * gh[:, 2 * _NF:])
    new_ref[...] = (1.0 - z) * n + z * out


def _update(agg2, cnt2, out, root, biasr, wih_t, bihr, whh_t, bhhr):
    bn = 2000
    return pl.pallas_call(
        _update_body,
        grid=(_N // bn,),
        in_specs=[
            pl.BlockSpec((2, bn, _NF), lambda i: (0, i, 0)),
            pl.BlockSpec((2, bn, _CW), lambda i: (0, i, 0)),
            pl.BlockSpec((bn, _NF), lambda i: (i, 0)),
            pl.BlockSpec((_NF, _NF), lambda i: (0, 0)),
            pl.BlockSpec((1, _NF), lambda i: (0, 0)),
            pl.BlockSpec((_NF, 3 * _NF), lambda i: (0, 0)),
            pl.BlockSpec((1, 3 * _NF), lambda i: (0, 0)),
            pl.BlockSpec((_NF, 3 * _NF), lambda i: (0, 0)),
            pl.BlockSpec((1, 3 * _NF), lambda i: (0, 0)),
        ],
        out_specs=pl.BlockSpec((bn, _NF), lambda i: (i, 0)),
        out_shape=jax.ShapeDtypeStruct((_N, _NF), jnp.float32),
    )(agg2, cnt2, out, root, biasr, wih_t, bihr, whh_t, bhhr)


def _sc_gather(table, src):
    @functools.partial(
        pl.kernel,
        mesh=_mesh,
        compiler_params=_SC_PARAMS,
        out_type=jax.ShapeDtypeStruct((_E, _NF), jnp.float32),
        scratch_types=[
            pltpu.VMEM((_CH,), jnp.int32),
            pltpu.VMEM((_CH,), jnp.int32),
            pltpu.VMEM((_CH, _NF), jnp.float32),
            pltpu.VMEM((_CH, _NF), jnp.float32),
            pltpu.SemaphoreType.DMA,
            pltpu.SemaphoreType.DMA,
            pltpu.SemaphoreType.DMA,
            pltpu.SemaphoreType.DMA,
            pltpu.SemaphoreType.DMA,
            pltpu.SemaphoreType.DMA,
        ],
    )
    def k(table_hbm, src_hbm, xj_hbm, i0, i1, r0, r1,
          si0, si1, sg0, sg1, ss0, ss1):
        wid = lax.axis_index("s") * 2 + lax.axis_index("c")
        idx = [i0, i1]
        rows = [r0, r1]
        sis = [si0, si1]
        sgs = [sg0, sg1]
        sss = [ss0, ss1]

        def base(j):
            return (wid + j * _NW) * _CH

        hi = [None] * _GT
        hs = [None] * _GT
        hi[0] = pltpu.async_copy(src_hbm.at[pl.ds(base(0), _CH)], idx[0],
                                 sis[0])
        for j in range(_GT):
            if j + 1 < _GT:
                b = (j + 1) % 2
                hi[j + 1] = pltpu.async_copy(
                    src_hbm.at[pl.ds(base(j + 1), _CH)], idx[b], sis[b])
            hi[j].wait()
            if j >= 2:
                hs[j - 2].wait()
            pltpu.async_copy(table_hbm.at[idx[j % 2]], rows[j % 2],
                             sgs[j % 2]).wait()
            hs[j] = pltpu.async_copy(rows[j % 2],
                                     xj_hbm.at[pl.ds(base(j), _CH)],
                                     sss[j % 2])
        hs[_GT - 2].wait()
        hs[_GT - 1].wait()

    return k(table, src)


def _make_sc_scatter(with_counts):
    outs = [jax.ShapeDtypeStruct((2, _N, _NF), jnp.float32)]
    scratch = [
        pltpu.VMEM((_CH,), jnp.int32),
        pltpu.VMEM((_CH,), jnp.int32),
        pltpu.VMEM((_CH, _NF), jnp.float32),
        pltpu.VMEM((_CH, _NF), jnp.float32),
        pltpu.VMEM_SHARED((_N, _NF), jnp.float32),
        pltpu.SemaphoreType.DMA,
        pltpu.SemaphoreType.DMA,
        pltpu.SemaphoreType.DMA,
        pltpu.SemaphoreType.DMA,
    ]
    if with_counts:
        outs.append(jax.ShapeDtypeStruct((2, _N, _CW), jnp.float32))
        scratch.append(pltpu.VMEM((_CH, _CW), jnp.float32))
        scratch.append(pltpu.VMEM_SHARED((_N, _CW), jnp.float32))

    def body(*refs):
        if with_counts:
            (msg_hbm, dst_hbm, z32_hbm, z16_hbm, ones_hbm, agg2_hbm,
             cnt2_hbm, i0, i1, v0, v1, agg_s, si0, si1, sv0, sv1,
             ones_v, cnt_s) = refs
        else:
            (msg_hbm, dst_hbm, z32_hbm, z16_hbm, ones_hbm, agg2_hbm,
             i0, i1, v0, v1, agg_s, si0, si1, sv0, sv1) = refs
        cid = lax.axis_index("c")
        sid = lax.axis_index("s")
        row0 = sid * _RPT
        idx = [i0, i1]
        val = [v0, v1]
        sis = [si0, si1]
        svs = [sv0, sv1]

        def base(j):
            return (cid * _CPS + sid + j * 16) * _CH

        pltpu.sync_copy(z32_hbm, agg_s.at[pl.ds(row0, _RPT)])
        if with_counts:
            pltpu.sync_copy(z16_hbm, cnt_s.at[pl.ds(row0, _RPT)])
            pltpu.sync_copy(ones_hbm, ones_v)
        plsc.subcore_barrier()

        hi = [None] * _ST
        hv = [None] * _ST
        hi[0] = pltpu.async_copy(dst_hbm.at[pl.ds(base(0), _CH)], idx[0],
                                 sis[0])
        hv[0] = pltpu.async_copy(msg_hbm.at[pl.ds(base(0), _CH)], val[0],
                                 svs[0])
        for j in range(_ST):
            if j + 1 < _ST:
                b = (j + 1) % 2
                hi[j + 1] = pltpu.async_copy(
                    dst_hbm.at[pl.ds(base(j + 1), _CH)], idx[b], sis[b])
                hv[j + 1] = pltpu.async_copy(
                    msg_hbm.at[pl.ds(base(j + 1), _CH)], val[b], svs[b])
            hi[j].wait()
            hv[j].wait()
            pltpu.sync_copy(val[j % 2], agg_s.at[idx[j % 2]], add=True)
            if with_counts:
                pltpu.sync_copy(ones_v, cnt_s.at[idx[j % 2]], add=True)

        plsc.subcore_barrier()
        pltpu.sync_copy(agg_s.at[pl.ds(row0, _RPT)],
                        agg2_hbm.at[cid, pl.ds(row0, _RPT)])
        if with_counts:
            pltpu.sync_copy(cnt_s.at[pl.ds(row0, _RPT)],
                            cnt2_hbm.at[cid, pl.ds(row0, _RPT)])

    return functools.partial(
        pl.kernel, mesh=_mesh,
        compiler_params=_SC_PARAMS,
        out_type=tuple(outs) if with_counts else outs[0],
        scratch_types=scratch,
    )(body)


_sc_scatter_counts = _make_sc_scatter(True)
_sc_scatter = _make_sc_scatter(False)


def kernel(h, edge_index, edge_weight, edge_attr, W0, b0, Ws, bs,
           W1, b1, W2, b2, root, bias, W_ih, W_hh, b_ih, b_hh):
    src = edge_index[0]
    dst = edge_index[1]

    eye4 = jnp.eye(4, dtype=jnp.float32)
    w2core = W2.reshape(_NH, _NF, _NF).transpose(1, 0, 2).reshape(
        _NF, _NH * _NF)
    w2a = jnp.kron(eye4, w2core)
    w2b = jnp.kron(eye4, b2.reshape(_NF, _NF))
    rmat = jnp.kron(jnp.eye(_NH, dtype=jnp.float32),
                    jnp.ones((1, _NF), jnp.float32))
    smat = jnp.kron(jnp.ones((_NH, 1), jnp.float32),
                    jnp.eye(_NF, dtype=jnp.float32))
    r4 = jnp.kron(eye4, rmat)
    s4 = jnp.kron(eye4, smat)
    wsbd = jnp.kron(eye4, Ws)
    w1bd = jnp.kron(eye4, W1)
    bs4 = jnp.tile(bs, 4).reshape(1, -1)
    b14 = jnp.tile(b1, 4).reshape(1, -1)
    ea4 = edge_attr.reshape(_E // 4, 4 * _DE)
    wih_t = W_ih.T
    whh_t = W_hh.T
    b0r = b0.reshape(1, -1)
    bsr = bs.reshape(1, -1)
    b1r = b1.reshape(1, -1)
    biasr = bias.reshape(1, -1)
    bihr = b_ih.reshape(1, -1)
    bhhr = b_hh.reshape(1, -1)
    z32 = jnp.zeros((_RPT, _NF), jnp.float32)
    z16 = jnp.zeros((_RPT, _CW), jnp.float32)
    o16 = jnp.ones((_CH, _CW), jnp.float32)

    out = _node_proj(h, W0, b0r)

    cnt2 = None
    for it in range(2):
        xjp = _sc_gather(out, src).reshape(_E // 4, 128)
        msg = _msg(xjp, ea4, wsbd, bs4, w1bd, b14,
                   w2a, w2b, r4, s4).reshape(_E, _NF)
        if it == 0:
            agg2, cnt2 = _sc_scatter_counts(msg, dst, z32, z16, o16)
        else:
            agg2 = _sc_scatter(msg, dst, z32, z16, o16)
        out = _update(agg2, cnt2, out, root, biasr, wih_t, bihr, whh_t, bhhr)
    return out
